# Initial kernel scaffold; baseline (speedup 1.0000x reference)
import jax
import jax.numpy as jnp
from jax.experimental import pallas as pl


def kernel(f2e_emb, local_entity, batch_heads, batch_rels, batch_tails, edge_mats, score_mats, rel_features, eps, W_kb, b_kb):
    raise NotImplementedError("write your pallas kernel here")


# speedup vs baseline: 1.0161x; 1.0161x over previous
import functools

import jax
import jax.numpy as jnp
from jax.experimental import pallas as pl
from jax.experimental.pallas import tpu as pltpu

K_EXTN = 2048
BLK = 512


def _samp_body(x_ref, xs_ref, g_ref, pos_ref, out_ref, *, N):
    b = pl.program_id(0)
    x = x_ref[0]
    xs = xs_ref[0]
    sq = jnp.sum(x * x, axis=1)
    sqs = jnp.sum(xs * xs, axis=1)
    dots = jax.lax.dot_general(xs, x, (((1,), (1,)), ((), ())),
                               precision=jax.lax.Precision.HIGHEST,
                               preferred_element_type=jnp.float32)
    d2 = jnp.maximum(sqs[:, None] + sq[None, :] - 2.0 * dots, 0.0)
    z = -jnp.sqrt(d2 + 1e-12)
    M = z + g_ref[0]
    m = jnp.max(M, axis=1, keepdims=True)
    iota = jax.lax.broadcasted_iota(jnp.int32, M.shape, 1)
    pidx = jnp.min(jnp.where(M == m, iota, N), axis=1)
    cols = pos_ref[0] & (N - 1)
    out_ref[0] = b * (N * N) + pidx[None, :] * N + cols


def _sampling(f2e_emb, xs, G, pos):
    B, N, d = f2e_emb.shape
    nb = K_EXTN // BLK
    pos3 = pos.reshape(B * nb, 1, BLK)
    out = pl.pallas_call(
        functools.partial(_samp_body, N=N),
        grid=(B, nb),
        in_specs=[
            pl.BlockSpec((1, N, d), lambda b, j: (b, 0, 0)),
            pl.BlockSpec((1, BLK, d), lambda b, j: (b, j, 0)),
            pl.BlockSpec((1, BLK, N), lambda b, j: (b, j, 0)),
            pl.BlockSpec((1, 1, BLK), lambda b, j, _nb=nb: (b * _nb + j, 0, 0)),
        ],
        out_specs=pl.BlockSpec((1, 1, BLK), lambda b, j, _nb=nb: (b * _nb + j, 0, 0)),
        out_shape=jax.ShapeDtypeStruct((B * nb, 1, BLK), jnp.int32),
    )(f2e_emb, xs, G, pos3)
    return out.reshape(B, K_EXTN)


def _ttable_body(rel_ref, w_ref, b_ref, out_ref, *, R):
    t = jax.lax.dot_general(rel_ref[...], w_ref[...], (((1,), (0,)), ((), ())),
                            precision=jax.lax.Precision.HIGHEST,
                            preferred_element_type=jnp.float32) + b_ref[...]
    rid = jax.lax.broadcasted_iota(jnp.int32, t.shape, 0)
    out_ref[...] = jnp.where(rid < R, t, 0.0)


def _ttable(rel_features, W_kb, b_kb):
    R, d = rel_features.shape
    Rp = K_EXTN
    relpad = jnp.zeros((Rp, d), jnp.float32).at[:R].set(rel_features)
    return pl.pallas_call(
        functools.partial(_ttable_body, R=R),
        out_shape=jax.ShapeDtypeStruct((Rp, d), jnp.float32),
    )(relpad, W_kb, b_kb[None, :])


def kernel(f2e_emb, local_entity, batch_heads, batch_rels, batch_tails, edge_mats, score_mats, rel_features, eps, W_kb, b_kb):
    B, N, d = f2e_emb.shape
    E = batch_heads.shape[1]
    R = rel_features.shape[0]
    key = jax.random.key(42)

    Tz = _ttable(rel_features, W_kb, b_kb)

    mask = (edge_mats == 0).reshape(B, N * N)
    n_valid = jnp.sum(mask, axis=1).astype(jnp.int32)
    pos = jax.vmap(lambda m: jnp.nonzero(m, size=K_EXTN, fill_value=0)[0])(mask)
    pos = pos.astype(jnp.int32)
    rows = pos >> 10
    cols = pos & (N - 1)

    xs = jnp.take_along_axis(f2e_emb, rows[:, :, None], axis=1)

    G = jnp.stack([
        jax.random.gumbel(jax.random.fold_in(key, i), (K_EXTN, N), jnp.float32)
        for i in range(B)])

    gidx = _sampling(f2e_emb, xs, G, pos)

    new_rels = edge_mats.reshape(-1)[gidx.reshape(-1)].reshape(B, K_EXTN)
    new_rels = new_rels.astype(jnp.int32)
    valid = (jnp.arange(K_EXTN)[None, :] < n_valid[:, None]) & (new_rels != 0)
    rel_ext = jnp.where(valid, new_rels, R)

    noff = (jnp.arange(B, dtype=jnp.int32) * N)[:, None]
    all_rels = jnp.concatenate([batch_rels.astype(jnp.int32), rel_ext], axis=1).reshape(-1)
    all_tails = jnp.concatenate([batch_tails.astype(jnp.int32), cols + noff], axis=1).reshape(-1)
    all_heads = jnp.concatenate([batch_heads.astype(jnp.int32), rows + noff], axis=1).reshape(-1)
    fv = Tz[all_rels]
    agg = jnp.zeros((B * N, d), jnp.float32)
    agg = agg.at[all_tails].add(fv)
    agg = agg.at[all_heads].add(fv)
    return jax.nn.relu(agg).reshape(B, N, d)


# speedup vs baseline: 1.2272x; 1.2078x over previous
import functools

import jax
import jax.numpy as jnp
from jax import lax
from jax.experimental import pallas as pl
from jax.experimental.pallas import tpu as pltpu
from jax.experimental.pallas import tpu_sc as plsc

K_EXTN = 2048
BLK = 512
CK = 128


def _samp_body(x_ref, xs_ref, g_ref, pos_ref, out_ref, *, N):
    b = pl.program_id(0)
    x = x_ref[0]
    xs = xs_ref[0]
    sq = jnp.sum(x * x, axis=1)
    sqs = jnp.sum(xs * xs, axis=1)
    dots = jax.lax.dot_general(xs, x, (((1,), (1,)), ((), ())),
                               precision=jax.lax.Precision.HIGHEST,
                               preferred_element_type=jnp.float32)
    d2 = jnp.maximum(sqs[:, None] + sq[None, :] - 2.0 * dots, 0.0)
    z = -jnp.sqrt(d2 + 1e-12)
    M = z + g_ref[0]
    m = jnp.max(M, axis=1, keepdims=True)
    iota = jax.lax.broadcasted_iota(jnp.int32, M.shape, 1)
    pidx = jnp.min(jnp.where(M == m, iota, N), axis=1)
    cols = pos_ref[0] & (N - 1)
    out_ref[0] = b * (N * N) + pidx[None, :] * N + cols


def _sampling(f2e_emb, xs, G, pos):
    B, N, d = f2e_emb.shape
    nb = K_EXTN // BLK
    pos3 = pos.reshape(B * nb, 1, BLK)
    out = pl.pallas_call(
        functools.partial(_samp_body, N=N),
        grid=(B, nb),
        in_specs=[
            pl.BlockSpec((1, N, d), lambda b, j: (b, 0, 0)),
            pl.BlockSpec((1, BLK, d), lambda b, j: (b, j, 0)),
            pl.BlockSpec((1, BLK, N), lambda b, j: (b, j, 0)),
            pl.BlockSpec((1, 1, BLK), lambda b, j, _nb=nb: (b * _nb + j, 0, 0)),
        ],
        out_specs=pl.BlockSpec((1, 1, BLK), lambda b, j, _nb=nb: (b * _nb + j, 0, 0)),
        out_shape=jax.ShapeDtypeStruct((B * nb, 1, BLK), jnp.int32),
    )(f2e_emb, xs, G, pos3)
    return out.reshape(B, K_EXTN)


def _ttable_body(rel_ref, w_ref, b_ref, out_ref, *, R):
    t = jax.lax.dot_general(rel_ref[...], w_ref[...], (((1,), (0,)), ((), ())),
                            precision=jax.lax.Precision.HIGHEST,
                            preferred_element_type=jnp.float32) + b_ref[...]
    rid = jax.lax.broadcasted_iota(jnp.int32, t.shape, 0)
    out_ref[...] = jnp.where(rid < R, t, 0.0)


def _ttable(rel_features, W_kb, b_kb):
    R, d = rel_features.shape
    Rp = K_EXTN
    relpad = jnp.zeros((Rp, d), jnp.float32).at[:R].set(rel_features)
    return pl.pallas_call(
        functools.partial(_ttable_body, R=R),
        out_shape=jax.ShapeDtypeStruct((Rp, d), jnp.float32),
    )(relpad, W_kb, b_kb[None, :])


def _agg_sc(tz, rels, tails, heads, zeros, BN, d):
    NT = rels.shape[0]
    nw = 32
    per_w = NT // nw
    nch = per_w // CK
    assert per_w * nw == NT and nch * CK == per_w
    rpt = BN // 16
    mesh = plsc.VectorSubcoreMesh(core_axis_name="c", subcore_axis_name="s")

    @functools.partial(
        pl.kernel,
        out_type=jax.ShapeDtypeStruct((2, BN, d), jnp.float32),
        mesh=mesh,
        scratch_types=[
            pltpu.VMEM((CK,), jnp.int32),
            pltpu.VMEM((CK,), jnp.int32),
            pltpu.VMEM((CK,), jnp.int32),
            pltpu.VMEM((CK, d), jnp.float32),
            pltpu.VMEM_SHARED((BN, d), jnp.float32),
            pltpu.SemaphoreType.DMA,
        ],
    )
    def k(tz_h, rel_h, tail_h, head_h, zero_h, out_h, relv, tailv, headv, rowsv, aggsh, sem):
        c = lax.axis_index("c")
        s = lax.axis_index("s")
        w = c * 16 + s
        pltpu.sync_copy(zero_h.at[pl.ds(s * rpt, rpt)], aggsh.at[pl.ds(s * rpt, rpt)])
        plsc.subcore_barrier()
        base0 = w * per_w

        def body(i, carry):
            base = base0 + i * CK
            pltpu.sync_copy(rel_h.at[pl.ds(base, CK)], relv)
            pltpu.sync_copy(tail_h.at[pl.ds(base, CK)], tailv)
            pltpu.sync_copy(head_h.at[pl.ds(base, CK)], headv)
            pltpu.async_copy(tz_h.at[relv], rowsv, sem).wait()
            pltpu.sync_copy(rowsv, aggsh.at[tailv], add=True)
            pltpu.sync_copy(rowsv, aggsh.at[headv], add=True)
            return carry

        lax.fori_loop(0, nch, body, 0)
        plsc.subcore_barrier()
        pltpu.sync_copy(aggsh.at[pl.ds(s * rpt, rpt)],
                        out_h.at[c, pl.ds(s * rpt, rpt)])

    return k(tz, rels, tails, heads, zeros)


def _combine_body(p_ref, out_ref):
    out_ref[...] = jnp.maximum(p_ref[0] + p_ref[1], 0.0)


def _combine(partials, BN, d):
    RB = 512
    return pl.pallas_call(
        _combine_body,
        grid=(BN // RB,),
        in_specs=[pl.BlockSpec((2, RB, d), lambda i: (0, i, 0))],
        out_specs=pl.BlockSpec((RB, d), lambda i: (i, 0)),
        out_shape=jax.ShapeDtypeStruct((BN, d), jnp.float32),
    )(partials)


def kernel(f2e_emb, local_entity, batch_heads, batch_rels, batch_tails, edge_mats, score_mats, rel_features, eps, W_kb, b_kb):
    B, N, d = f2e_emb.shape
    E = batch_heads.shape[1]
    R = rel_features.shape[0]
    key = jax.random.key(42)

    Tz = _ttable(rel_features, W_kb, b_kb)

    mask = (edge_mats == 0).reshape(B, N * N)
    n_valid = jnp.sum(mask, axis=1).astype(jnp.int32)
    pos = jax.vmap(lambda m: jnp.nonzero(m, size=K_EXTN, fill_value=0)[0])(mask)
    pos = pos.astype(jnp.int32)
    rows = pos >> 10
    cols = pos & (N - 1)

    xs = jnp.take_along_axis(f2e_emb, rows[:, :, None], axis=1)

    G = jnp.stack([
        jax.random.gumbel(jax.random.fold_in(key, i), (K_EXTN, N), jnp.float32)
        for i in range(B)])

    gidx = _sampling(f2e_emb, xs, G, pos)

    new_rels = edge_mats.reshape(-1)[gidx.reshape(-1)].reshape(B, K_EXTN)
    new_rels = new_rels.astype(jnp.int32)
    valid = (jnp.arange(K_EXTN)[None, :] < n_valid[:, None]) & (new_rels != 0)
    rel_ext = jnp.where(valid, new_rels, R)

    noff = (jnp.arange(B, dtype=jnp.int32) * N)[:, None]
    all_rels = jnp.concatenate([batch_rels.astype(jnp.int32), rel_ext], axis=1).reshape(-1)
    all_tails = jnp.concatenate([batch_tails.astype(jnp.int32), cols + noff], axis=1).reshape(-1)
    all_heads = jnp.concatenate([batch_heads.astype(jnp.int32), rows + noff], axis=1).reshape(-1)
    zeros = jnp.zeros((B * N, d), jnp.float32)
    partials = _agg_sc(Tz, all_rels, all_tails, all_heads, zeros, B * N, d)
    return _combine(partials, B * N, d).reshape(B, N, d)


# speedup vs baseline: 2.7373x; 2.2305x over previous
import functools

import jax
import jax.numpy as jnp
from jax import lax
from jax.experimental import pallas as pl
from jax.experimental.pallas import tpu as pltpu
from jax.experimental.pallas import tpu_sc as plsc

K_EXTN = 2048
BLK = 512
CK = 128


def _samp_body(x_ref, xs_ref, g_ref, pos_ref, out_ref, *, N):
    b = pl.program_id(0)
    x = x_ref[0]
    xs = xs_ref[0]
    sq = jnp.sum(x * x, axis=1)
    sqs = jnp.sum(xs * xs, axis=1)
    dots = jax.lax.dot_general(xs, x, (((1,), (1,)), ((), ())),
                               precision=jax.lax.Precision.HIGHEST,
                               preferred_element_type=jnp.float32)
    d2 = jnp.maximum(sqs[:, None] + sq[None, :] - 2.0 * dots, 0.0)
    z = -jnp.sqrt(d2 + 1e-12)
    M = z + g_ref[0]
    m = jnp.max(M, axis=1, keepdims=True)
    iota = jax.lax.broadcasted_iota(jnp.int32, M.shape, 1)
    pidx = jnp.min(jnp.where(M == m, iota, N), axis=1)
    cols = pos_ref[0] & (N - 1)
    out_ref[0] = b * (N * N) + pidx[None, :] * N + cols


def _sampling(f2e_emb, xs, G, pos):
    B, N, d = f2e_emb.shape
    nb = K_EXTN // BLK
    pos3 = pos.reshape(B * nb, 1, BLK)
    out = pl.pallas_call(
        functools.partial(_samp_body, N=N),
        grid=(B, nb),
        in_specs=[
            pl.BlockSpec((1, N, d), lambda b, j: (b, 0, 0)),
            pl.BlockSpec((1, BLK, d), lambda b, j: (b, j, 0)),
            pl.BlockSpec((1, BLK, N), lambda b, j: (b, j, 0)),
            pl.BlockSpec((1, 1, BLK), lambda b, j, _nb=nb: (b * _nb + j, 0, 0)),
        ],
        out_specs=pl.BlockSpec((1, 1, BLK), lambda b, j, _nb=nb: (b * _nb + j, 0, 0)),
        out_shape=jax.ShapeDtypeStruct((B * nb, 1, BLK), jnp.int32),
    )(f2e_emb, xs, G, pos3)
    return out.reshape(B, K_EXTN)


def _ttable_body(rel_ref, w_ref, b_ref, out_ref, *, R):
    t = jax.lax.dot_general(rel_ref[...], w_ref[...], (((1,), (0,)), ((), ())),
                            precision=jax.lax.Precision.HIGHEST,
                            preferred_element_type=jnp.float32) + b_ref[...]
    rid = jax.lax.broadcasted_iota(jnp.int32, t.shape, 0)
    out_ref[...] = jnp.where(rid < R, t, 0.0)


def _ttable(rel_features, W_kb, b_kb):
    R, d = rel_features.shape
    Rp = K_EXTN
    relpad = jnp.zeros((Rp, d), jnp.float32).at[:R].set(rel_features)
    return pl.pallas_call(
        functools.partial(_ttable_body, R=R),
        out_shape=jax.ShapeDtypeStruct((Rp, d), jnp.float32),
    )(relpad, W_kb, b_kb[None, :])


def _compact_sc(em2, f2e_flat, B, N, d):
    NN = N * N
    CH = NN // 16
    SUB = 8192
    nsub = CH // SUB
    K = K_EXTN
    mesh = plsc.VectorSubcoreMesh(core_axis_name="c", subcore_axis_name="s")

    @functools.partial(
        pl.kernel,
        out_type=(
            jax.ShapeDtypeStruct((B, K), jnp.int32),
            jax.ShapeDtypeStruct((B, 16), jnp.int32),
            jax.ShapeDtypeStruct((B, K, d), jnp.float32),
        ),
        mesh=mesh,
        scratch_types=[
            pltpu.VMEM((SUB,), jnp.int32),
            pltpu.VMEM((K,), jnp.int32),
            pltpu.VMEM((16, K // 16), jnp.int32),
            pltpu.VMEM((16,), jnp.int32),
            pltpu.VMEM((K // 16,), jnp.int32),
            pltpu.VMEM((K // 16,), jnp.int32),
            pltpu.VMEM((K // 16, 128), jnp.float32),
            pltpu.VMEM((K // 16,), jnp.int32),
            pltpu.VMEM_SHARED((16, 16), jnp.int32),
            pltpu.VMEM_SHARED((16, 16, K // 16), jnp.int32),
            pltpu.SemaphoreType.DMA,
        ],
        compiler_params=pltpu.CompilerParams(needs_layout_passes=False),
    )
    def k(em_h, f2e_h, pos_h, nval_h, xs_h, subv, locb, outl, cnt16, posv,
          rowv, xsr, rowb, cnts_sh, slabs_sh, sem):
        c = lax.axis_index("c")
        s = lax.axis_index("s")
        lane = lax.iota(jnp.int32, 16)
        zero16 = jnp.zeros((16,), jnp.int32)
        KS = K // 16
        KSH = KS.bit_length() - 1

        for u in range(B // 2):
            b = c * (B // 2) + u
            for i in range(16):
                for jz in range(KS // 16):
                    outl[i, pl.ds(jz * 16, 16)] = zero16

            def sub_body(j, cntv):
                pltpu.sync_copy(em_h.at[b, pl.ds(s * CH + j * SUB, SUB)], subv)

                def vec_body(i, cntv):
                    v = subv[pl.ds(i * 16, 16)]
                    m = v == 0
                    pref = jnp.cumsum(jnp.where(m, 1, 0))
                    tgt = cntv + pref - 1
                    p = (s * CH + lane + i * 16) + j * SUB
                    plsc.store_scatter(locb, (tgt,), p,
                                       mask=m & (tgt < K))
                    return cntv + plsc.all_reduce_population_count(m)

                return lax.fori_loop(0, SUB // 16, vec_body, cntv)

            cntv = lax.fori_loop(0, nsub, sub_body, zero16)

            cnt16[...] = cntv
            pltpu.sync_copy(cnt16, cnts_sh.at[s])
            plsc.subcore_barrier()

            basev = zero16
            totv = zero16
            for t in range(16):
                pltpu.sync_copy(cnts_sh.at[t], cnt16)
                row = cnt16[...]
                basev = basev + jnp.where(t < s, row, 0)
                totv = totv + row

            def place_body(i, carry):
                jl = lane + i * 16
                tg = basev + jl
                val = locb[pl.ds(i * 16, 16)]
                plsc.store_scatter(outl, (jnp.right_shift(tg, KSH), tg & (KS - 1)),
                                   val, mask=(jl < cntv) & (tg < K))
                return carry

            lax.fori_loop(0, K // 16, place_body, 0)
            pltpu.sync_copy(outl, slabs_sh.at[s])
            plsc.subcore_barrier()

            acc = [zero16] * (KS // 16)
            for t in range(16):
                pltpu.sync_copy(slabs_sh.at[t, s], rowb)
                for q in range(KS // 16):
                    acc[q] = acc[q] + rowb[pl.ds(q * 16, 16)]
            for q in range(KS // 16):
                posv[pl.ds(q * 16, 16)] = acc[q]

            pltpu.sync_copy(posv, pos_h.at[b, pl.ds(s * KS, KS)])
            sh = N.bit_length() - 1
            for i in range(KS // 16):
                pv = posv[pl.ds(i * 16, 16)]
                rowv[pl.ds(i * 16, 16)] = jnp.right_shift(pv, sh) + b * N
            pltpu.async_copy(f2e_h.at[rowv], xsr, sem).wait()
            pltpu.sync_copy(xsr, xs_h.at[b, pl.ds(s * KS, KS), :])

            @pl.when(s == 0)
            def _():
                cnt16[...] = totv
                pltpu.sync_copy(cnt16, nval_h.at[b])

            plsc.subcore_barrier()

    return k(em2, f2e_flat)


def _agg_sc(tz, rels, tails, heads, zeros, BN, d):
    NT = rels.shape[0]
    nw = 32
    per_w = NT // nw
    nch = per_w // CK
    assert per_w * nw == NT and nch * CK == per_w
    rpt = BN // 16
    mesh = plsc.VectorSubcoreMesh(core_axis_name="c", subcore_axis_name="s")

    @functools.partial(
        pl.kernel,
        out_type=jax.ShapeDtypeStruct((2, BN, d), jnp.float32),
        mesh=mesh,
        scratch_types=[
            pltpu.VMEM((CK,), jnp.int32),
            pltpu.VMEM((CK,), jnp.int32),
            pltpu.VMEM((CK,), jnp.int32),
            pltpu.VMEM((CK, d), jnp.float32),
            pltpu.VMEM_SHARED((BN, d), jnp.float32),
            pltpu.SemaphoreType.DMA,
        ],
    )
    def k(tz_h, rel_h, tail_h, head_h, zero_h, out_h, relv, tailv, headv, rowsv, aggsh, sem):
        c = lax.axis_index("c")
        s = lax.axis_index("s")
        w = c * 16 + s
        pltpu.sync_copy(zero_h.at[pl.ds(s * rpt, rpt)], aggsh.at[pl.ds(s * rpt, rpt)])
        plsc.subcore_barrier()
        base0 = w * per_w

        def body(i, carry):
            base = base0 + i * CK
            pltpu.sync_copy(rel_h.at[pl.ds(base, CK)], relv)
            pltpu.sync_copy(tail_h.at[pl.ds(base, CK)], tailv)
            pltpu.sync_copy(head_h.at[pl.ds(base, CK)], headv)
            pltpu.async_copy(tz_h.at[relv], rowsv, sem).wait()
            pltpu.sync_copy(rowsv, aggsh.at[tailv], add=True)
            pltpu.sync_copy(rowsv, aggsh.at[headv], add=True)
            return carry

        lax.fori_loop(0, nch, body, 0)
        plsc.subcore_barrier()
        pltpu.sync_copy(aggsh.at[pl.ds(s * rpt, rpt)],
                        out_h.at[c, pl.ds(s * rpt, rpt)])

    return k(tz, rels, tails, heads, zeros)


def _combine_body(p_ref, out_ref):
    out_ref[...] = jnp.maximum(p_ref[0] + p_ref[1], 0.0)


def _combine(partials, BN, d):
    RB = 512
    return pl.pallas_call(
        _combine_body,
        grid=(BN // RB,),
        in_specs=[pl.BlockSpec((2, RB, d), lambda i: (0, i, 0))],
        out_specs=pl.BlockSpec((RB, d), lambda i: (i, 0)),
        out_shape=jax.ShapeDtypeStruct((BN, d), jnp.float32),
    )(partials)


def kernel(f2e_emb, local_entity, batch_heads, batch_rels, batch_tails, edge_mats, score_mats, rel_features, eps, W_kb, b_kb):
    B, N, d = f2e_emb.shape
    E = batch_heads.shape[1]
    R = rel_features.shape[0]
    key = jax.random.key(42)

    Tz = _ttable(rel_features, W_kb, b_kb)

    em2 = edge_mats.reshape(B, N * N).astype(jnp.int32)
    pos, nval16, xs = _compact_sc(em2, f2e_emb.reshape(B * N, d), B, N, d)
    n_valid = nval16[:, 0]
    rows = pos >> (N.bit_length() - 1)
    cols = pos & (N - 1)

    G = jnp.stack([
        jax.random.gumbel(jax.random.fold_in(key, i), (K_EXTN, N), jnp.float32)
        for i in range(B)])

    gidx = _sampling(f2e_emb, xs, G, pos)

    new_rels = edge_mats.reshape(-1)[gidx.reshape(-1)].reshape(B, K_EXTN)
    new_rels = new_rels.astype(jnp.int32)
    valid = (jnp.arange(K_EXTN)[None, :] < n_valid[:, None]) & (new_rels != 0)
    rel_ext = jnp.where(valid, new_rels, R)

    noff = (jnp.arange(B, dtype=jnp.int32) * N)[:, None]
    all_rels = jnp.concatenate([batch_rels.astype(jnp.int32), rel_ext], axis=1).reshape(-1)
    all_tails = jnp.concatenate([batch_tails.astype(jnp.int32), cols + noff], axis=1).reshape(-1)
    all_heads = jnp.concatenate([batch_heads.astype(jnp.int32), rows + noff], axis=1).reshape(-1)
    zeros = jnp.zeros((B * N, d), jnp.float32)
    partials = _agg_sc(Tz, all_rels, all_tails, all_heads, zeros, B * N, d)
    return _combine(partials, B * N, d).reshape(B, N, d)


# speedup vs baseline: 2.7833x; 1.0168x over previous
import functools

import jax
import jax.numpy as jnp
from jax import lax
from jax.experimental import pallas as pl
from jax.experimental.pallas import tpu as pltpu
from jax.experimental.pallas import tpu_sc as plsc

K_EXTN = 2048
BLK = 512
CK = 128


def _samp_body(x_ref, xs_ref, g_ref, pos_ref, out_ref, *, N):
    b = pl.program_id(0)
    x = x_ref[0]
    xs = xs_ref[0]
    sq = jnp.sum(x * x, axis=1)
    sqs = jnp.sum(xs * xs, axis=1)
    dots = jax.lax.dot_general(xs, x, (((1,), (1,)), ((), ())),
                               precision=jax.lax.Precision.HIGHEST,
                               preferred_element_type=jnp.float32)
    d2 = jnp.maximum(sqs[:, None] + sq[None, :] - 2.0 * dots, 0.0)
    z = -jnp.sqrt(d2 + 1e-12)
    M = z + g_ref[0]
    m = jnp.max(M, axis=1, keepdims=True)
    iota = jax.lax.broadcasted_iota(jnp.int32, M.shape, 1)
    pidx = jnp.min(jnp.where(M == m, iota, N), axis=1)
    cols = pos_ref[0] & (N - 1)
    out_ref[0] = b * (N * N) + pidx[None, :] * N + cols


def _sampling(f2e_emb, xs, G, pos):
    B, N, d = f2e_emb.shape
    nb = K_EXTN // BLK
    pos3 = pos.reshape(B * nb, 1, BLK)
    out = pl.pallas_call(
        functools.partial(_samp_body, N=N),
        grid=(B, nb),
        in_specs=[
            pl.BlockSpec((1, N, d), lambda b, j: (b, 0, 0)),
            pl.BlockSpec((1, BLK, d), lambda b, j: (b, j, 0)),
            pl.BlockSpec((1, BLK, N), lambda b, j: (b, j, 0)),
            pl.BlockSpec((1, 1, BLK), lambda b, j, _nb=nb: (b * _nb + j, 0, 0)),
        ],
        out_specs=pl.BlockSpec((1, 1, BLK), lambda b, j, _nb=nb: (b * _nb + j, 0, 0)),
        out_shape=jax.ShapeDtypeStruct((B * nb, 1, BLK), jnp.int32),
    )(f2e_emb, xs, G, pos3)
    return out.reshape(B, K_EXTN)


def _ttable_body(rel_ref, w_ref, b_ref, out_ref, *, R):
    t = jax.lax.dot_general(rel_ref[...], w_ref[...], (((1,), (0,)), ((), ())),
                            precision=jax.lax.Precision.HIGHEST,
                            preferred_element_type=jnp.float32) + b_ref[...]
    rid = jax.lax.broadcasted_iota(jnp.int32, t.shape, 0)
    out_ref[...] = jnp.where(rid < R, t, 0.0)


def _ttable(rel_features, W_kb, b_kb):
    R, d = rel_features.shape
    Rp = K_EXTN
    relpad = jnp.zeros((Rp, d), jnp.float32).at[:R].set(rel_features)
    return pl.pallas_call(
        functools.partial(_ttable_body, R=R),
        out_shape=jax.ShapeDtypeStruct((Rp, d), jnp.float32),
    )(relpad, W_kb, b_kb[None, :])


def _compact_sc(em2, f2e_flat, B, N, d):
    NN = N * N
    CH = NN // 16
    SUB = 8192
    nsub = CH // SUB
    K = K_EXTN
    mesh = plsc.VectorSubcoreMesh(core_axis_name="c", subcore_axis_name="s")

    @functools.partial(
        pl.kernel,
        out_type=(
            jax.ShapeDtypeStruct((B, K), jnp.int32),
            jax.ShapeDtypeStruct((B, 16), jnp.int32),
            jax.ShapeDtypeStruct((B, K, d), jnp.float32),
        ),
        mesh=mesh,
        scratch_types=[
            pltpu.VMEM((SUB,), jnp.int32),
            pltpu.VMEM((K,), jnp.int32),
            pltpu.VMEM((16, K // 16), jnp.int32),
            pltpu.VMEM((16,), jnp.int32),
            pltpu.VMEM((K // 16,), jnp.int32),
            pltpu.VMEM((K // 16,), jnp.int32),
            pltpu.VMEM((K // 16, 128), jnp.float32),
            pltpu.VMEM((K // 16,), jnp.int32),
            pltpu.VMEM_SHARED((16, 16), jnp.int32),
            pltpu.VMEM_SHARED((16, 16, K // 16), jnp.int32),
            pltpu.SemaphoreType.DMA,
        ],
        compiler_params=pltpu.CompilerParams(needs_layout_passes=False),
    )
    def k(em_h, f2e_h, pos_h, nval_h, xs_h, subv, locb, outl, cnt16, posv,
          rowv, xsr, rowb, cnts_sh, slabs_sh, sem):
        c = lax.axis_index("c")
        s = lax.axis_index("s")
        lane = lax.iota(jnp.int32, 16)
        zero16 = jnp.zeros((16,), jnp.int32)
        KS = K // 16
        KSH = KS.bit_length() - 1

        for u in range(B // 2):
            b = c * (B // 2) + u
            for i in range(16):
                for jz in range(KS // 16):
                    outl[i, pl.ds(jz * 16, 16)] = zero16

            def sub_body(j, cntv):
                pltpu.sync_copy(em_h.at[b, pl.ds(s * CH + j * SUB, SUB)], subv)

                def vec_body(i, cntv):
                    v = subv[pl.ds(i * 16, 16)]
                    m = v == 0
                    pref = jnp.cumsum(jnp.where(m, 1, 0))
                    tgt = cntv + pref - 1
                    p = (s * CH + lane + i * 16) + j * SUB
                    plsc.store_scatter(locb, (tgt,), p,
                                       mask=m & (tgt < K))
                    return cntv + plsc.all_reduce_population_count(m)

                return lax.fori_loop(0, SUB // 16, vec_body, cntv)

            cntv = lax.fori_loop(0, nsub, sub_body, zero16)

            cnt16[...] = cntv
            pltpu.sync_copy(cnt16, cnts_sh.at[s])
            plsc.subcore_barrier()

            basev = zero16
            totv = zero16
            for t in range(16):
                pltpu.sync_copy(cnts_sh.at[t], cnt16)
                row = cnt16[...]
                basev = basev + jnp.where(t < s, row, 0)
                totv = totv + row

            def place_body(i, carry):
                jl = lane + i * 16
                tg = basev + jl
                val = locb[pl.ds(i * 16, 16)]
                plsc.store_scatter(outl, (jnp.right_shift(tg, KSH), tg & (KS - 1)),
                                   val, mask=(jl < cntv) & (tg < K))
                return carry

            lax.fori_loop(0, K // 16, place_body, 0)
            pltpu.sync_copy(outl, slabs_sh.at[s])
            plsc.subcore_barrier()

            acc = [zero16] * (KS // 16)
            for t in range(16):
                pltpu.sync_copy(slabs_sh.at[t, s], rowb)
                for q in range(KS // 16):
                    acc[q] = acc[q] + rowb[pl.ds(q * 16, 16)]
            for q in range(KS // 16):
                posv[pl.ds(q * 16, 16)] = acc[q]

            pltpu.sync_copy(posv, pos_h.at[b, pl.ds(s * KS, KS)])
            sh = N.bit_length() - 1
            for i in range(KS // 16):
                pv = posv[pl.ds(i * 16, 16)]
                rowv[pl.ds(i * 16, 16)] = jnp.right_shift(pv, sh) + b * N
            pltpu.async_copy(f2e_h.at[rowv], xsr, sem).wait()
            pltpu.sync_copy(xsr, xs_h.at[b, pl.ds(s * KS, KS), :])

            @pl.when(s == 0)
            def _():
                cnt16[...] = totv
                pltpu.sync_copy(cnt16, nval_h.at[b])

            plsc.subcore_barrier()

    return k(em2, f2e_flat)


def _agg_sc(tz, rels, tails, heads, zeros, BN, d):
    NT = rels.shape[0]
    nw = 32
    per_w = NT // nw
    nch = per_w // CK
    assert per_w * nw == NT and nch * CK == per_w
    rpt = BN // 16
    rel2 = rels.reshape(nw, nch, CK)
    tail2 = tails.reshape(nw, nch, CK)
    head2 = heads.reshape(nw, nch, CK)
    mesh = plsc.VectorSubcoreMesh(core_axis_name="c", subcore_axis_name="s")

    @functools.partial(
        pl.kernel,
        out_type=jax.ShapeDtypeStruct((2, BN, d), jnp.float32),
        mesh=mesh,
        scratch_types=[
            pltpu.VMEM((nch, CK), jnp.int32),
            pltpu.VMEM((nch, CK), jnp.int32),
            pltpu.VMEM((nch, CK), jnp.int32),
            pltpu.VMEM((2, CK, d), jnp.float32),
            pltpu.VMEM_SHARED((BN, d), jnp.float32),
            pltpu.SemaphoreType.DMA,
            pltpu.SemaphoreType.DMA,
            pltpu.SemaphoreType.DMA,
            pltpu.SemaphoreType.DMA,
        ],
    )
    def k(tz_h, rel_h, tail_h, head_h, zero_h, out_h, relv, tailv, headv,
          rowsv, aggsh, semz, semi, semg0, semg1):
        c = lax.axis_index("c")
        s = lax.axis_index("s")
        w = c * 16 + s
        zcp = pltpu.async_copy(zero_h.at[pl.ds(s * rpt, rpt)],
                               aggsh.at[pl.ds(s * rpt, rpt)], semz)
        i0 = pltpu.async_copy(rel_h.at[w], relv, semi)
        i1 = pltpu.async_copy(tail_h.at[w], tailv, semi)
        i2 = pltpu.async_copy(head_h.at[w], headv, semi)
        i0.wait(); i1.wait(); i2.wait()
        zcp.wait()
        plsc.subcore_barrier()

        semg = [semg0, semg1]
        gat = [None, None]
        sca = []
        for i in range(nch):
            pb = i % 2
            if i == 0:
                gat[0] = pltpu.async_copy(tz_h.at[relv.at[0]], rowsv.at[0], semg[0])
            if i >= 1:
                for dsc in sca:
                    dsc.wait()
                sca = []
            gat[pb].wait()
            if i + 1 < nch:
                gat[1 - pb] = pltpu.async_copy(
                    tz_h.at[relv.at[i + 1]], rowsv.at[1 - pb], semg[1 - pb])
            sca.append(pltpu.async_copy(rowsv.at[pb], aggsh.at[tailv.at[i]],
                                        semg[pb], add=True))
            sca.append(pltpu.async_copy(rowsv.at[pb], aggsh.at[headv.at[i]],
                                        semg[pb], add=True))
        for dsc in sca:
            dsc.wait()
        plsc.subcore_barrier()
        pltpu.sync_copy(aggsh.at[pl.ds(s * rpt, rpt)],
                        out_h.at[c, pl.ds(s * rpt, rpt)])

    return k(tz, rel2, tail2, head2, zeros)


def _combine_body(p_ref, out_ref):
    out_ref[...] = jnp.maximum(p_ref[0] + p_ref[1], 0.0)


def _combine(partials, BN, d):
    RB = 512
    return pl.pallas_call(
        _combine_body,
        grid=(BN // RB,),
        in_specs=[pl.BlockSpec((2, RB, d), lambda i: (0, i, 0))],
        out_specs=pl.BlockSpec((RB, d), lambda i: (i, 0)),
        out_shape=jax.ShapeDtypeStruct((BN, d), jnp.float32),
    )(partials)


def kernel(f2e_emb, local_entity, batch_heads, batch_rels, batch_tails, edge_mats, score_mats, rel_features, eps, W_kb, b_kb):
    B, N, d = f2e_emb.shape
    E = batch_heads.shape[1]
    R = rel_features.shape[0]
    key = jax.random.key(42)

    Tz = _ttable(rel_features, W_kb, b_kb)

    em2 = edge_mats.reshape(B, N * N).astype(jnp.int32)
    pos, nval16, xs = _compact_sc(em2, f2e_emb.reshape(B * N, d), B, N, d)
    n_valid = nval16[:, 0]
    rows = pos >> (N.bit_length() - 1)
    cols = pos & (N - 1)

    G = jnp.stack([
        jax.random.gumbel(jax.random.fold_in(key, i), (K_EXTN, N), jnp.float32)
        for i in range(B)])

    gidx = _sampling(f2e_emb, xs, G, pos)

    new_rels = edge_mats.reshape(-1)[gidx.reshape(-1)].reshape(B, K_EXTN)
    new_rels = new_rels.astype(jnp.int32)
    valid = (jnp.arange(K_EXTN)[None, :] < n_valid[:, None]) & (new_rels != 0)
    rel_ext = jnp.where(valid, new_rels, R)

    noff = (jnp.arange(B, dtype=jnp.int32) * N)[:, None]
    all_rels = jnp.concatenate([batch_rels.astype(jnp.int32), rel_ext], axis=1).reshape(-1)
    all_tails = jnp.concatenate([batch_tails.astype(jnp.int32), cols + noff], axis=1).reshape(-1)
    all_heads = jnp.concatenate([batch_heads.astype(jnp.int32), rows + noff], axis=1).reshape(-1)
    zeros = jnp.zeros((B * N, d), jnp.float32)
    partials = _agg_sc(Tz, all_rels, all_tails, all_heads, zeros, B * N, d)
    return _combine(partials, B * N, d).reshape(B, N, d)
